# baseline (speedup 1.0000x reference)
import functools

import jax
import jax.numpy as jnp
from jax import lax
from jax.experimental import pallas as pl
from jax.experimental.pallas import tpu as pltpu
from jax.experimental.pallas import tpu_sc as plsc

_B = 16384
_F = 32
_L = 16

_INFO = plsc.get_sparse_core_info()
_NC = _INFO.num_cores
_NS = _INFO.num_subcores
_NW = _NC * _NS
_BPW = _B // _NW
_CHUNK = 128
_NCHUNK = _BPW // _CHUNK

_mesh = plsc.VectorSubcoreMesh(core_axis_name="c", subcore_axis_name="s")


@functools.partial(
    pl.kernel,
    out_type=jax.ShapeDtypeStruct((_B,), jnp.float32),
    mesh=_mesh,
    compiler_params=pltpu.CompilerParams(needs_layout_passes=False,
                                         use_tc_tiling_on_sc=False),
    scratch_types=[
        pltpu.VMEM((_NCHUNK, _CHUNK), jnp.int32),
        pltpu.VMEM((_NCHUNK, _CHUNK), jnp.int32),
        pltpu.VMEM((_BPW, _F), jnp.float32),
        pltpu.VMEM((_BPW, _F), jnp.float32),
        pltpu.VMEM((_BPW,), jnp.float32),
        pltpu.VMEM((_BPW,), jnp.float32),
        pltpu.VMEM((_BPW,), jnp.float32),
        pltpu.SemaphoreType.DMA,
    ],
)
def _mf_kernel(uid_hbm, iid_hbm, p_hbm, q_hbm, ub_hbm, ib_hbm, out_hbm,
               uidx_v, iidx_v, prow_v, qrow_v, ubias_v, ibias_v, out_v, sem):
    wid = lax.axis_index("s") * _NC + lax.axis_index("c")
    base = wid * _BPW

    for j in range(_NCHUNK):
        pltpu.sync_copy(uid_hbm.at[pl.ds(base + j * _CHUNK, _CHUNK)],
                        uidx_v.at[j])
        pltpu.sync_copy(iid_hbm.at[pl.ds(base + j * _CHUNK, _CHUNK)],
                        iidx_v.at[j])

    copies = []
    for j in range(_NCHUNK):
        rs = pl.ds(j * _CHUNK, _CHUNK)
        copies.append(pltpu.async_copy(p_hbm.at[uidx_v.at[j]], prow_v.at[rs], sem))
        copies.append(pltpu.async_copy(q_hbm.at[iidx_v.at[j]], qrow_v.at[rs], sem))
        copies.append(pltpu.async_copy(ub_hbm.at[uidx_v.at[j]], ubias_v.at[rs], sem))
        copies.append(pltpu.async_copy(ib_hbm.at[iidx_v.at[j]], ibias_v.at[rs], sem))
    for c in copies:
        c.wait()

    lane = lax.iota(jnp.int32, _L)

    def block(b, carry):
        sl = pl.ds(b * _L, _L)
        acc = ubias_v[sl] + ibias_v[sl]
        for j in range(_L):
            r = b * _L + j
            p0 = prow_v[r, pl.ds(0, _L)]
            p1 = prow_v[r, pl.ds(_L, _L)]
            q0 = qrow_v[r, pl.ds(0, _L)]
            q1 = qrow_v[r, pl.ds(_L, _L)]
            s = jnp.sum(p0 * q0 + p1 * q1, axis=0)
            acc = jnp.where(lane == j, acc + s, acc)
        out_v[sl] = acc
        return carry

    lax.fori_loop(0, _BPW // _L, block, 0)

    pltpu.sync_copy(out_v, out_hbm.at[pl.ds(base, _BPW)])


def kernel(user_id, item_id, P, Q, user_bias, item_bias):
    return _mf_kernel(user_id.astype(jnp.int32), item_id.astype(jnp.int32),
                      P, Q,
                      user_bias.reshape(-1), item_bias.reshape(-1))
